# baseline (speedup 1.0000x reference)
import jax
import jax.numpy as jnp
from jax.experimental import pallas as pl
from jax.experimental.pallas import tpu as pltpu
from jax.experimental.pallas import tpu_sc as plsc

_LANES = 16


def _sc_add(patch2d, pos_table, num_batches):
    n, d = pos_table.shape
    rows, _ = patch2d.shape
    r = 16
    n_tiles = n // r
    mesh = plsc.VectorSubcoreMesh(core_axis_name="c", subcore_axis_name="s")

    @pl.kernel(
        out_type=jax.ShapeDtypeStruct(patch2d.shape, patch2d.dtype),
        mesh=mesh,
    )
    def sc_kernel(p_hbm, t_hbm, o_hbm):
        def body(p_ref, t_ref, o_ref):
            for i in range(r):
                for c in range(0, d, _LANES):
                    slc = (pl.ds(i, 1), pl.ds(c, _LANES))
                    o_ref.at[*slc][...] = p_ref.at[*slc][...] + t_ref.at[*slc][...]

        pltpu.emit_pipeline(
            body,
            grid=(num_batches, n_tiles),
            in_specs=[
                pl.BlockSpec((r, d), index_map=lambda b, i: (b * n_tiles + i, 0)),
                pl.BlockSpec((r, d), index_map=lambda b, i: (i, 0)),
            ],
            out_specs=[
                pl.BlockSpec((r, d), index_map=lambda b, i: (b * n_tiles + i, 0)),
            ],
            core_axis_name=("c", "s"),
            dimension_semantics=(pltpu.PARALLEL, pltpu.ARBITRARY),
        )(p_hbm, t_hbm, o_hbm)

    return sc_kernel(patch2d, pos_table)


def kernel(patch, pos_table):
    batch, num_patches, proj_dim = patch.shape
    patch2d = patch.reshape(batch * num_patches, proj_dim)
    out2d = _sc_add(patch2d, pos_table, batch)
    return out2d.reshape(batch, num_patches, proj_dim)


# speedup vs baseline: 2.5435x; 2.5435x over previous
import jax
import jax.numpy as jnp
from jax.experimental import pallas as pl
from jax.experimental.pallas import tpu as pltpu
from jax.experimental.pallas import tpu_sc as plsc

_LANES = 16


def _sc_add(patch2d, pos_table, num_batches):
    n, d = pos_table.shape
    rows, _ = patch2d.shape
    r = 16
    n_tiles = n // r
    mesh = plsc.VectorSubcoreMesh(core_axis_name="c", subcore_axis_name="s")

    @pl.kernel(
        out_type=jax.ShapeDtypeStruct(patch2d.shape, patch2d.dtype),
        mesh=mesh,
    )
    def sc_kernel(p_hbm, t_hbm, o_hbm):
        def body(p_ref, t_ref, o_ref):
            @plsc.parallel_loop(0, r, unroll=2)
            def _(i):
                for c in range(0, d, _LANES):
                    slc = (pl.ds(i, 1), pl.ds(c, _LANES))
                    o_ref.at[*slc][...] = p_ref.at[*slc][...] + t_ref.at[*slc][...]

        pltpu.emit_pipeline(
            body,
            grid=(num_batches, n_tiles),
            in_specs=[
                pl.BlockSpec((r, d), index_map=lambda b, i: (b * n_tiles + i, 0)),
                pl.BlockSpec((r, d), index_map=lambda b, i: (i, 0)),
            ],
            out_specs=[
                pl.BlockSpec((r, d), index_map=lambda b, i: (b * n_tiles + i, 0)),
            ],
            core_axis_name=("c", "s"),
            dimension_semantics=(pltpu.PARALLEL, pltpu.ARBITRARY),
        )(p_hbm, t_hbm, o_hbm)

    return sc_kernel(patch2d, pos_table)


def kernel(patch, pos_table):
    batch, num_patches, proj_dim = patch.shape
    patch2d = patch.reshape(batch * num_patches, proj_dim)
    out2d = _sc_add(patch2d, pos_table, batch)
    return out2d.reshape(batch, num_patches, proj_dim)
